# baseline (speedup 1.0000x reference)
import functools

import jax
import jax.numpy as jnp
from jax import lax
from jax.experimental import pallas as pl
from jax.experimental.pallas import tpu as pltpu

_B, _T, _D = 4, 8192, 2048
_H1, _E = 256, 64
_ROWS = 1024
_KPB = _T // _ROWS


def _body(x_ref, w1_ref, b1_ref, w2_ref, b2_ref, w_ref, i_ref, acc_ref):
    b = pl.program_id(0)
    k = pl.program_id(1)

    row = lax.broadcasted_iota(jnp.int32, (8, 1), 0)
    psum = jnp.sum(x_ref[0], axis=0, keepdims=True)
    part = jnp.where(row % _B == b, psum, 0.0)

    @pl.when(jnp.logical_and(b == 0, k == 0))
    def _init():
        acc_ref[...] = part

    @pl.when(jnp.logical_or(b > 0, k > 0))
    def _acc():
        acc_ref[...] += part

    @pl.when(jnp.logical_and(b == _B - 1, k == _KPB - 1))
    def _gate():
        xm = acc_ref[0:_B, :] * (1.0 / _T)
        h = jnp.maximum(
            jnp.dot(xm, w1_ref[...], preferred_element_type=jnp.float32)
            + b1_ref[...], 0.0)
        g = (jnp.dot(h, w2_ref[...], preferred_element_type=jnp.float32)
             + b2_ref[...])
        gmax = jnp.max(g, axis=-1, keepdims=True)
        e = jnp.exp(g - gmax)
        p = e / jnp.sum(e, axis=-1, keepdims=True)
        iota = lax.broadcasted_iota(jnp.int32, p.shape, 1)
        v1 = jnp.max(p, axis=-1, keepdims=True)
        i1 = jnp.min(jnp.where(p == v1, iota, _E), axis=-1, keepdims=True)
        p2 = jnp.where(iota == i1, -jnp.inf, p)
        v2 = jnp.max(p2, axis=-1, keepdims=True)
        i2 = jnp.min(jnp.where(p2 == v2, iota, _E), axis=-1, keepdims=True)
        s = v1 + v2
        w_ref[...] = jnp.concatenate([v1 / s, v2 / s], axis=1)
        i_ref[...] = jnp.concatenate([i1, i2], axis=1)


@functools.partial(jax.jit, static_argnames=("interpret",))
def _run(x, W1, b1, W2, b2, interpret=False):
    w, idx = pl.pallas_call(
        _body,
        grid=(_B, _KPB),
        in_specs=[
            pl.BlockSpec((1, _ROWS, _D), lambda b, k: (b, k, 0)),
            pl.BlockSpec((_D, _H1), lambda b, k: (0, 0)),
            pl.BlockSpec((1, _H1), lambda b, k: (0, 0)),
            pl.BlockSpec((_H1, _E), lambda b, k: (0, 0)),
            pl.BlockSpec((1, _E), lambda b, k: (0, 0)),
        ],
        out_specs=[
            pl.BlockSpec((_B, 2), lambda b, k: (0, 0)),
            pl.BlockSpec((_B, 2), lambda b, k: (0, 0)),
        ],
        out_shape=[
            jax.ShapeDtypeStruct((_B, 2), jnp.float32),
            jax.ShapeDtypeStruct((_B, 2), jnp.int32),
        ],
        scratch_shapes=[pltpu.VMEM((8, _D), jnp.float32)],
        compiler_params=pltpu.CompilerParams(
            dimension_semantics=("arbitrary", "arbitrary")),
        interpret=interpret,
    )(x, W1, b1.reshape(1, _H1), W2, b2.reshape(1, _E))
    return w, idx


def kernel(x, W1, b1, W2, b2):
    return _run(x, W1, b1, W2, b2)


# speedup vs baseline: 1.0026x; 1.0026x over previous
import functools

import jax
import jax.numpy as jnp
from jax import lax
from jax.experimental import pallas as pl
from jax.experimental.pallas import tpu as pltpu

_B, _T, _D = 4, 8192, 2048
_H1, _E = 256, 64
_ROWS = 1024
_KPB = _T // _ROWS


def _body(x_ref, w1_ref, b1_ref, w2_ref, b2_ref, w_ref, i_ref,
          acc_ref, w1_v, b1_v, w2_v, b2_v, sem):
    b = pl.program_id(0)
    k = pl.program_id(1)

    @pl.when(jnp.logical_and(b == 0, k == 0))
    def _fetch_weights():
        pltpu.async_copy(w1_ref, w1_v, sem)
        pltpu.async_copy(b1_ref, b1_v, sem)
        pltpu.async_copy(w2_ref, w2_v, sem)
        pltpu.async_copy(b2_ref, b2_v, sem)

    row = lax.broadcasted_iota(jnp.int32, (8, 1), 0)
    psum = jnp.sum(x_ref[0], axis=0, keepdims=True)
    part = jnp.where(row % _B == b, psum, 0.0)

    @pl.when(jnp.logical_and(b == 0, k == 0))
    def _init():
        acc_ref[...] = part

    @pl.when(jnp.logical_or(b > 0, k > 0))
    def _acc():
        acc_ref[...] += part

    @pl.when(jnp.logical_and(b == _B - 1, k == _KPB - 1))
    def _gate():
        pltpu.make_async_copy(w1_ref, w1_v, sem).wait()
        pltpu.make_async_copy(b1_ref, b1_v, sem).wait()
        pltpu.make_async_copy(w2_ref, w2_v, sem).wait()
        pltpu.make_async_copy(b2_ref, b2_v, sem).wait()
        xm = acc_ref[0:_B, :] * (1.0 / _T)
        h = jnp.maximum(
            jnp.dot(xm, w1_v[...], preferred_element_type=jnp.float32)
            + b1_v[...], 0.0)
        g = (jnp.dot(h, w2_v[...], preferred_element_type=jnp.float32)
             + b2_v[...])
        gmax = jnp.max(g, axis=-1, keepdims=True)
        e = jnp.exp(g - gmax)
        p = e / jnp.sum(e, axis=-1, keepdims=True)
        iota = lax.broadcasted_iota(jnp.int32, p.shape, 1)
        v1 = jnp.max(p, axis=-1, keepdims=True)
        i1 = jnp.min(jnp.where(p == v1, iota, _E), axis=-1, keepdims=True)
        p2 = jnp.where(iota == i1, -jnp.inf, p)
        v2 = jnp.max(p2, axis=-1, keepdims=True)
        i2 = jnp.min(jnp.where(p2 == v2, iota, _E), axis=-1, keepdims=True)
        s = v1 + v2
        wv = jnp.concatenate([v1 / s, v2 / s], axis=1)
        iv = jnp.concatenate([i1, i2], axis=1)
        w_ref[...] = jnp.pad(wv, ((0, 4), (0, 126)))
        i_ref[...] = jnp.pad(iv, ((0, 4), (0, 126)))


@functools.partial(jax.jit, static_argnames=("interpret",))
def _run(x, W1, b1, W2, b2, interpret=False):
    wpad, ipad = pl.pallas_call(
        _body,
        grid=(_B, _KPB),
        in_specs=[
            pl.BlockSpec((1, _ROWS, _D), lambda b, k: (b, k, 0)),
            pl.BlockSpec(memory_space=pl.ANY),
            pl.BlockSpec(memory_space=pl.ANY),
            pl.BlockSpec(memory_space=pl.ANY),
            pl.BlockSpec(memory_space=pl.ANY),
        ],
        out_specs=[
            pl.BlockSpec((8, 128), lambda b, k: (0, 0)),
            pl.BlockSpec((8, 128), lambda b, k: (0, 0)),
        ],
        out_shape=[
            jax.ShapeDtypeStruct((8, 128), jnp.float32),
            jax.ShapeDtypeStruct((8, 128), jnp.int32),
        ],
        scratch_shapes=[
            pltpu.VMEM((8, _D), jnp.float32),
            pltpu.VMEM((_D, _H1), jnp.float32),
            pltpu.VMEM((1, _H1), jnp.float32),
            pltpu.VMEM((_H1, _E), jnp.float32),
            pltpu.VMEM((1, _E), jnp.float32),
            pltpu.SemaphoreType.DMA,
        ],
        compiler_params=pltpu.CompilerParams(
            dimension_semantics=("arbitrary", "arbitrary")),
        interpret=interpret,
    )(x, W1, b1.reshape(1, _H1), W2, b2.reshape(1, _E))
    return wpad[0:_B, 0:2], ipad[0:_B, 0:2]


def kernel(x, W1, b1, W2, b2):
    return _run(x, W1, b1, W2, b2)


# speedup vs baseline: 1.0518x; 1.0491x over previous
import functools

import jax
import jax.numpy as jnp
from jax import lax
from jax.experimental import pallas as pl
from jax.experimental.pallas import tpu as pltpu

_B, _T, _D = 4, 8192, 2048
_H1, _E = 256, 64
_ROWS = 1024
_KPB = _T // _ROWS


def _body(x_ref, w1_ref, b1_ref, w2_ref, b2_ref, w_ref, i_ref,
          acc_ref, w1_v, b1_v, w2_v, b2_v, sem):
    b = pl.program_id(0)
    k = pl.program_id(1)

    @pl.when(jnp.logical_and(b == 0, k == 0))
    def _fetch_weights():
        pltpu.async_copy(w1_ref, w1_v, sem)
        pltpu.async_copy(b1_ref, b1_v, sem)
        pltpu.async_copy(w2_ref, w2_v, sem)
        pltpu.async_copy(b2_ref, b2_v, sem)

    row = lax.broadcasted_iota(jnp.int32, (8, 1), 0)
    psum = jnp.sum(x_ref[0], axis=0, keepdims=True)
    part = jnp.where(row % _B == b, psum, 0.0)

    @pl.when(jnp.logical_and(b == 0, k == 0))
    def _init():
        acc_ref[...] = part

    @pl.when(jnp.logical_or(b > 0, k > 0))
    def _acc():
        acc_ref[...] += part

    @pl.when(jnp.logical_and(b == _B - 1, k == _KPB - 1))
    def _gate():
        pltpu.make_async_copy(w1_ref, w1_v, sem).wait()
        pltpu.make_async_copy(b1_ref, b1_v, sem).wait()
        pltpu.make_async_copy(w2_ref, w2_v, sem).wait()
        pltpu.make_async_copy(b2_ref, b2_v, sem).wait()
        xm = acc_ref[0:_B, :] * (1.0 / _T)
        h = jnp.maximum(
            jnp.dot(xm, w1_v[...], preferred_element_type=jnp.float32)
            + b1_v[...], 0.0)
        g = (lax.dot_general(h, w2_v[...], (((1,), (1,)), ((), ())),
                             preferred_element_type=jnp.float32)
             + b2_v[...])
        gmax = jnp.max(g, axis=-1, keepdims=True)
        e = jnp.exp(g - gmax)
        p = e / jnp.sum(e, axis=-1, keepdims=True)
        iota = lax.broadcasted_iota(jnp.int32, p.shape, 1)
        v1 = jnp.max(p, axis=-1, keepdims=True)
        i1 = jnp.min(jnp.where(p == v1, iota, _E), axis=-1, keepdims=True)
        p2 = jnp.where(iota == i1, -jnp.inf, p)
        v2 = jnp.max(p2, axis=-1, keepdims=True)
        i2 = jnp.min(jnp.where(p2 == v2, iota, _E), axis=-1, keepdims=True)
        s = v1 + v2
        wv = jnp.concatenate([v1 / s, v2 / s], axis=1)
        iv = jnp.concatenate([i1, i2], axis=1)
        w_ref[...] = jnp.pad(wv, ((0, 4), (0, 126)))
        i_ref[...] = jnp.pad(iv, ((0, 4), (0, 126)))


@functools.partial(jax.jit, static_argnames=("interpret",))
def _run(x, W1, b1, W2, b2, interpret=False):
    wpad, ipad = pl.pallas_call(
        _body,
        grid=(_B, _KPB),
        in_specs=[
            pl.BlockSpec((1, _ROWS, _D), lambda b, k: (b, k, 0)),
            pl.BlockSpec(memory_space=pl.ANY),
            pl.BlockSpec(memory_space=pl.ANY),
            pl.BlockSpec(memory_space=pl.ANY),
            pl.BlockSpec(memory_space=pl.ANY),
        ],
        out_specs=[
            pl.BlockSpec((8, 128), lambda b, k: (0, 0)),
            pl.BlockSpec((8, 128), lambda b, k: (0, 0)),
        ],
        out_shape=[
            jax.ShapeDtypeStruct((8, 128), jnp.float32),
            jax.ShapeDtypeStruct((8, 128), jnp.int32),
        ],
        scratch_shapes=[
            pltpu.VMEM((8, _D), jnp.float32),
            pltpu.VMEM((_D, _H1), jnp.float32),
            pltpu.VMEM((1, _H1), jnp.float32),
            pltpu.VMEM((_E, _H1), jnp.float32),
            pltpu.VMEM((1, _E), jnp.float32),
            pltpu.SemaphoreType.DMA,
        ],
        compiler_params=pltpu.CompilerParams(
            dimension_semantics=("arbitrary", "arbitrary")),
        interpret=interpret,
    )(x, W1, b1.reshape(1, _H1), W2.T, b2.reshape(1, _E))
    return wpad[0:_B, 0:2], ipad[0:_B, 0:2]


def kernel(x, W1, b1, W2, b2):
    return _run(x, W1, b1, W2, b2)
